# Initial kernel scaffold; baseline (speedup 1.0000x reference)
import functools

import jax
import jax.numpy as jnp
from jax import lax
from jax.experimental import pallas as pl
from jax.experimental.pallas import tpu as pltpu
from jax.experimental.pallas import tpu_sc as plsc

_N = 10000
_E = 320000
_IN = 128
_HID = 64
_OUT = 32
_ALPHA = 0.2
_BETA = 0.8

_NC, _NS = 2, 16
_NW = _NC * _NS
_CHUNK = 128
_CPW = -(-_E // (_NW * _CHUNK))
_EPAD = _NW * _CPW * _CHUNK
_NPAD = 10240
_STRIPE = _NPAD // _NS

_mesh = plsc.VectorSubcoreMesh(
    core_axis_name="c", subcore_axis_name="s", num_cores=_NC, num_subcores=_NS)


@functools.partial(
    pl.kernel,
    out_type=(jax.ShapeDtypeStruct((_NC, _NPAD), jnp.float32),
              jax.ShapeDtypeStruct((_NC, _NPAD), jnp.float32)),
    mesh=_mesh,
    scratch_types=[
        pltpu.VMEM((_CPW, _CHUNK), jnp.int32),
        pltpu.VMEM((_CPW, _CHUNK), jnp.int32),
        pltpu.VMEM((_CHUNK,), jnp.float32),
        pltpu.VMEM((_CHUNK,), jnp.float32),
        pltpu.VMEM_SHARED((_NPAD,), jnp.float32),
        pltpu.VMEM_SHARED((_NPAD,), jnp.float32),
    ],
)
def _sc_degrees(row_hbm, col_hbm, ideg_out, odeg_out,
                row_v, col_v, ones_v, zeros_v, iacc, oacc):
    c = lax.axis_index("c")
    s = lax.axis_index("s")
    wid = c * _NS + s

    def initbuf(g, carry):
        ones_v[pl.ds(g * 16, 16)] = jnp.ones((16,), jnp.float32)
        zeros_v[pl.ds(g * 16, 16)] = jnp.zeros((16,), jnp.float32)
        return carry
    lax.fori_loop(0, _CHUNK // 16, initbuf, None)

    def zloop(b, carry):
        off = s * _STRIPE + b * _CHUNK
        pltpu.sync_copy(zeros_v, iacc.at[pl.ds(off, _CHUNK)])
        pltpu.sync_copy(zeros_v, oacc.at[pl.ds(off, _CHUNK)])
        return carry
    lax.fori_loop(0, _STRIPE // _CHUNK, zloop, None)

    pltpu.sync_copy(row_hbm.at[wid], row_v)
    pltpu.sync_copy(col_hbm.at[wid], col_v)
    plsc.subcore_barrier()

    def body(i, carry):
        pltpu.sync_copy(ones_v, iacc.at[col_v.at[i]], add=True)
        pltpu.sync_copy(ones_v, oacc.at[row_v.at[i]], add=True)
        return carry
    lax.fori_loop(0, _CPW, body, None)

    plsc.subcore_barrier()
    sl = pl.ds(s * _STRIPE, _STRIPE)
    pltpu.sync_copy(iacc.at[sl], ideg_out.at[c, sl])
    pltpu.sync_copy(oacc.at[sl], odeg_out.at[c, sl])


def _make_sc_agg(D, with_sbe):
    out_type = [jax.ShapeDtypeStruct((_NC, _NPAD, D), jnp.float32)]
    scratch = [
        pltpu.VMEM((_CPW, _CHUNK), jnp.int32),
        pltpu.VMEM((_CPW, _CHUNK), jnp.int32),
        pltpu.VMEM((_CHUNK, D), jnp.float32),
        pltpu.VMEM((_CHUNK, D), jnp.float32),
        pltpu.VMEM_SHARED((_NPAD, D), jnp.float32),
        pltpu.SemaphoreType.DMA,
    ]
    if with_sbe:
        out_type.append(jax.ShapeDtypeStruct((_NC, _NPAD), jnp.float32))
        scratch += [
            pltpu.VMEM((_CHUNK,), jnp.float32),
            pltpu.VMEM_SHARED((_NPAD,), jnp.float32),
            pltpu.SemaphoreType.DMA,
        ]

    def body(*refs):
        if with_sbe:
            (feat_hbm, row_hbm, col_hbm, be_hbm, out_hbm, sbe_out,
             row_v, col_v, rows_v, zb, acc, sem, bev, sbe_acc, sem2) = refs
        else:
            (feat_hbm, row_hbm, col_hbm, out_hbm,
             row_v, col_v, rows_v, zb, acc, sem) = refs
        c = lax.axis_index("c")
        s = lax.axis_index("s")
        wid = c * _NS + s

        def zrow(r, carry):
            for g in range(D // 16):
                zb[r, pl.ds(g * 16, 16)] = jnp.zeros((16,), jnp.float32)
            return carry
        lax.fori_loop(0, _CHUNK, zrow, None)

        def zloop(b, carry):
            off = s * _STRIPE + b * _CHUNK
            pltpu.sync_copy(zb, acc.at[pl.ds(off, _CHUNK)])
            if with_sbe:
                pltpu.sync_copy(zb.at[0], sbe_acc.at[pl.ds(off, _CHUNK)])
            return carry
        lax.fori_loop(0, _STRIPE // _CHUNK, zloop, None)

        pltpu.sync_copy(row_hbm.at[wid], row_v)
        pltpu.sync_copy(col_hbm.at[wid], col_v)
        plsc.subcore_barrier()

        def body_i(i, carry):
            ri = row_v.at[i]
            pltpu.async_copy(feat_hbm.at[ri], rows_v, sem).wait()
            pltpu.sync_copy(rows_v, acc.at[col_v.at[i]], add=True)
            if with_sbe:
                pltpu.async_copy(be_hbm.at[ri], bev, sem2).wait()
                pltpu.sync_copy(bev, sbe_acc.at[col_v.at[i]], add=True)
            return carry
        lax.fori_loop(0, _CPW, body_i, None)

        plsc.subcore_barrier()
        sl = pl.ds(s * _STRIPE, _STRIPE)
        pltpu.sync_copy(acc.at[sl], out_hbm.at[c, sl])
        if with_sbe:
            pltpu.sync_copy(sbe_acc.at[sl], sbe_out.at[c, sl])

    return pl.kernel(body, out_type=tuple(out_type), mesh=_mesh,
                     scratch_types=scratch)


_sc_agg1 = _make_sc_agg(_IN, with_sbe=True)
_sc_agg2 = _make_sc_agg(_HID, with_sbe=False)


def _tc_prep_body(deg_ref, x_ref, a_ref, be_ref, xbe_ref):
    deg = deg_ref[...]
    ideg = deg[:, 0:1] + deg[:, 1:2] + 1.0
    odeg = deg[:, 2:3] + deg[:, 3:4] + 1.0
    a = jnp.exp(-_ALPHA * jnp.log(ideg))
    be = jnp.exp(-_BETA * jnp.log(odeg))
    a_ref[...] = a
    be_ref[...] = be
    xbe_ref[...] = x_ref[...] * be


_TCB = 1024


def _tc_prep(deg4, x_pad):
    grid = (_NPAD // _TCB,)
    return pl.pallas_call(
        _tc_prep_body,
        grid=grid,
        in_specs=[
            pl.BlockSpec((_TCB, 4), lambda i: (i, 0)),
            pl.BlockSpec((_TCB, _IN), lambda i: (i, 0)),
        ],
        out_specs=[
            pl.BlockSpec((_TCB, 1), lambda i: (i, 0)),
            pl.BlockSpec((_TCB, 1), lambda i: (i, 0)),
            pl.BlockSpec((_TCB, _IN), lambda i: (i, 0)),
        ],
        out_shape=[
            jax.ShapeDtypeStruct((_NPAD, 1), jnp.float32),
            jax.ShapeDtypeStruct((_NPAD, 1), jnp.float32),
            jax.ShapeDtypeStruct((_NPAD, _IN), jnp.float32),
        ],
    )(deg4, x_pad)


def _tc_mid_body(y1p_ref, xbe_ref, a_ref, be_ref, sbe_ref,
                 ws1t_ref, wt1t_ref, ws2t_ref, wt2t_ref, bs1_ref, bt1_ref,
                 zbe_ref, d1_ref):
    a = a_ref[...]
    be = be_ref[...]
    y1 = a * (y1p_ref[0] + y1p_ref[1] + xbe_ref[...])
    sbe = sbe_ref[...]
    d1 = a * (sbe[:, 0:1] + sbe[:, 1:2] + be)
    s_h = jnp.maximum(
        jnp.dot(y1, ws1t_ref[...], preferred_element_type=jnp.float32)
        + d1 * bs1_ref[...], 0.0)
    t_h = jnp.maximum(
        jnp.dot(y1, wt1t_ref[...], preferred_element_type=jnp.float32)
        + d1 * bt1_ref[...], 0.0)
    z = jnp.concatenate(
        [jnp.dot(t_h, ws2t_ref[...], preferred_element_type=jnp.float32),
         jnp.dot(s_h, wt2t_ref[...], preferred_element_type=jnp.float32)],
        axis=1)
    zbe_ref[...] = be * z
    d1_ref[...] = d1


def _tc_mid(y1p, xbe, a1, be1, sbeT, ws1t, wt1t, ws2t, wt2t, bs1, bt1):
    grid = (_NPAD // _TCB,)
    full = lambda i: (0, 0)
    return pl.pallas_call(
        _tc_mid_body,
        grid=grid,
        in_specs=[
            pl.BlockSpec((_NC, _TCB, _IN), lambda i: (0, i, 0)),
            pl.BlockSpec((_TCB, _IN), lambda i: (i, 0)),
            pl.BlockSpec((_TCB, 1), lambda i: (i, 0)),
            pl.BlockSpec((_TCB, 1), lambda i: (i, 0)),
            pl.BlockSpec((_TCB, 2), lambda i: (i, 0)),
            pl.BlockSpec((_IN, _HID), full),
            pl.BlockSpec((_IN, _HID), full),
            pl.BlockSpec((_HID, _OUT), full),
            pl.BlockSpec((_HID, _OUT), full),
            pl.BlockSpec((1, _HID), full),
            pl.BlockSpec((1, _HID), full),
        ],
        out_specs=[
            pl.BlockSpec((_TCB, _HID), lambda i: (i, 0)),
            pl.BlockSpec((_TCB, 1), lambda i: (i, 0)),
        ],
        out_shape=[
            jax.ShapeDtypeStruct((_NPAD, _HID), jnp.float32),
            jax.ShapeDtypeStruct((_NPAD, 1), jnp.float32),
        ],
    )(y1p, xbe, a1, be1, sbeT, ws1t, wt1t, ws2t, wt2t, bs1, bt1)


def _tc_final_body(y2p_ref, zbe_ref, a_ref, d1_ref, bcat_ref, out_ref):
    out_ref[...] = (a_ref[...] * (y2p_ref[0] + y2p_ref[1] + zbe_ref[...])
                    + d1_ref[...] * bcat_ref[...])


def _tc_final(y2p, zbe, a1, d1, bcat):
    grid = (_NPAD // _TCB,)
    return pl.pallas_call(
        _tc_final_body,
        grid=grid,
        in_specs=[
            pl.BlockSpec((_NC, _TCB, _HID), lambda i: (0, i, 0)),
            pl.BlockSpec((_TCB, _HID), lambda i: (i, 0)),
            pl.BlockSpec((_TCB, 1), lambda i: (i, 0)),
            pl.BlockSpec((_TCB, 1), lambda i: (i, 0)),
            pl.BlockSpec((1, _HID), lambda i: (0, 0)),
        ],
        out_specs=pl.BlockSpec((_TCB, _HID), lambda i: (i, 0)),
        out_shape=jax.ShapeDtypeStruct((_NPAD, _HID), jnp.float32),
    )(y2p, zbe, a1, d1, bcat)


def kernel(x, edge_index, edge_attr, W_s1, b_s1, W_t1, b_t1,
           W_s2, b_s2, W_t2, b_t2):
    row = edge_index[0]
    col = edge_index[1]
    npad = _EPAD - _E
    dummy = jnp.full((npad,), _NPAD - 1, jnp.int32)
    row3 = jnp.concatenate([row, dummy]).reshape(_NW, _CPW, _CHUNK)
    col3 = jnp.concatenate([col, dummy]).reshape(_NW, _CPW, _CHUNK)

    ideg_p, odeg_p = _sc_degrees(row3, col3)
    deg4 = jnp.stack([ideg_p[0], ideg_p[1], odeg_p[0], odeg_p[1]], axis=1)
    x_pad = jnp.pad(x, ((0, _NPAD - _N), (0, 0)))
    a1, be1, xbe = _tc_prep(deg4, x_pad)

    y1p, sbep = _sc_agg1(xbe, row3, col3, be1.reshape(_NPAD))
    sbeT = jnp.transpose(sbep)

    zbe, d1 = _tc_mid(y1p, xbe, a1, be1, sbeT,
                      W_s1.T, W_t1.T, W_s2.T, W_t2.T,
                      b_s1[None, :], b_t1[None, :])

    (y2p,) = _sc_agg2(zbe, row3, col3)

    bcat = jnp.concatenate([b_s2, b_t2])[None, :]
    outp = _tc_final(y2p, zbe, a1, d1, bcat)
    return outp[:_N]


# speedup vs baseline: 17.0741x; 17.0741x over previous
import functools

import jax
import jax.numpy as jnp
from jax import lax
from jax.experimental import pallas as pl
from jax.experimental.pallas import tpu as pltpu
from jax.experimental.pallas import tpu_sc as plsc

_N = 10000
_E = 320000
_IN = 128
_HID = 64
_OUT = 32
_ALPHA = 0.2
_BETA = 0.8

_NC, _NS = 2, 16
_NW = _NC * _NS
_CHUNK = 128
_CPW = -(-_E // (_NW * _CHUNK))
_EPAD = _NW * _CPW * _CHUNK
_NPAD = 10240
_STRIPE = _NPAD // _NS

_mesh = plsc.VectorSubcoreMesh(
    core_axis_name="c", subcore_axis_name="s", num_cores=_NC, num_subcores=_NS)


@functools.partial(
    pl.kernel,
    out_type=(jax.ShapeDtypeStruct((_NC, _NPAD), jnp.float32),
              jax.ShapeDtypeStruct((_NC, _NPAD), jnp.float32)),
    mesh=_mesh,
    scratch_types=[
        pltpu.VMEM((_CPW, _CHUNK), jnp.int32),
        pltpu.VMEM((_CPW, _CHUNK), jnp.int32),
        pltpu.VMEM((_CHUNK,), jnp.float32),
        pltpu.VMEM((_CHUNK,), jnp.float32),
        pltpu.VMEM_SHARED((_NPAD,), jnp.float32),
        pltpu.VMEM_SHARED((_NPAD,), jnp.float32),
    ],
)
def _sc_degrees(row_hbm, col_hbm, ideg_out, odeg_out,
                row_v, col_v, ones_v, zeros_v, iacc, oacc):
    c = lax.axis_index("c")
    s = lax.axis_index("s")
    wid = c * _NS + s

    def initbuf(g, carry):
        ones_v[pl.ds(g * 16, 16)] = jnp.ones((16,), jnp.float32)
        zeros_v[pl.ds(g * 16, 16)] = jnp.zeros((16,), jnp.float32)
        return carry
    lax.fori_loop(0, _CHUNK // 16, initbuf, None)

    def zloop(b, carry):
        off = s * _STRIPE + b * _CHUNK
        pltpu.sync_copy(zeros_v, iacc.at[pl.ds(off, _CHUNK)])
        pltpu.sync_copy(zeros_v, oacc.at[pl.ds(off, _CHUNK)])
        return carry
    lax.fori_loop(0, _STRIPE // _CHUNK, zloop, None)

    pltpu.sync_copy(row_hbm.at[wid], row_v)
    pltpu.sync_copy(col_hbm.at[wid], col_v)
    plsc.subcore_barrier()

    def body(i, carry):
        pltpu.sync_copy(ones_v, iacc.at[col_v.at[i]], add=True)
        pltpu.sync_copy(ones_v, oacc.at[row_v.at[i]], add=True)
        return carry
    lax.fori_loop(0, _CPW, body, None)

    plsc.subcore_barrier()
    sl = pl.ds(s * _STRIPE, _STRIPE)
    pltpu.sync_copy(iacc.at[sl], ideg_out.at[c, sl])
    pltpu.sync_copy(oacc.at[sl], odeg_out.at[c, sl])


def _make_sc_agg(with_sbe):
    D = _IN
    out_type = [jax.ShapeDtypeStruct((_NC, _NPAD, D), jnp.float32)]
    scratch = [
        pltpu.VMEM((_CPW, _CHUNK), jnp.int32),
        pltpu.VMEM((_CPW, _CHUNK), jnp.int32),
        pltpu.VMEM((_CHUNK, D), jnp.float32),
        pltpu.VMEM_SHARED((_NPAD, D), jnp.float32),
        pltpu.SemaphoreType.DMA,
    ]
    if with_sbe:
        out_type.append(jax.ShapeDtypeStruct((_NC, _NPAD), jnp.float32))
        scratch += [
            pltpu.VMEM((_CHUNK,), jnp.float32),
            pltpu.VMEM_SHARED((_NPAD,), jnp.float32),
            pltpu.SemaphoreType.DMA,
        ]

    def body(*refs):
        if with_sbe:
            (feat_hbm, row_hbm, col_hbm, be_hbm, out_hbm, sbe_out,
             row_v, col_v, rows_v, acc, sem, bev, sbe_acc, sem2) = refs
        else:
            (feat_hbm, row_hbm, col_hbm, out_hbm,
             row_v, col_v, rows_v, acc, sem) = refs
        c = lax.axis_index("c")
        s = lax.axis_index("s")
        wid = c * _NS + s

        def zrow(r, carry):
            for g in range(D // 16):
                rows_v[r, pl.ds(g * 16, 16)] = jnp.zeros((16,), jnp.float32)
            return carry
        lax.fori_loop(0, _CHUNK, zrow, None)

        def zloop(b, carry):
            off = s * _STRIPE + b * _CHUNK
            pltpu.sync_copy(rows_v, acc.at[pl.ds(off, _CHUNK)])
            if with_sbe:
                pltpu.sync_copy(rows_v.at[0], sbe_acc.at[pl.ds(off, _CHUNK)])
            return carry
        lax.fori_loop(0, _STRIPE // _CHUNK, zloop, None)

        pltpu.sync_copy(row_hbm.at[wid], row_v)
        pltpu.sync_copy(col_hbm.at[wid], col_v)
        plsc.subcore_barrier()

        def body_i(i, carry):
            ri = row_v.at[i]
            pltpu.async_copy(feat_hbm.at[ri], rows_v, sem).wait()
            pltpu.sync_copy(rows_v, acc.at[col_v.at[i]], add=True)
            if with_sbe:
                pltpu.async_copy(be_hbm.at[ri], bev, sem2).wait()
                pltpu.sync_copy(bev, sbe_acc.at[col_v.at[i]], add=True)
            return carry
        lax.fori_loop(0, _CPW, body_i, None)

        plsc.subcore_barrier()
        sl = pl.ds(s * _STRIPE, _STRIPE)
        pltpu.sync_copy(acc.at[sl], out_hbm.at[c, sl])
        if with_sbe:
            pltpu.sync_copy(sbe_acc.at[sl], sbe_out.at[c, sl])

    return pl.kernel(body, out_type=tuple(out_type), mesh=_mesh,
                     scratch_types=scratch)


_sc_agg1 = _make_sc_agg(with_sbe=True)
_sc_agg2 = _make_sc_agg(with_sbe=False)


def _tc_prep_body(deg_ref, x_ref, a_ref, be_ref, xbe_ref):
    deg = deg_ref[...]
    ideg = deg[:, 0:1] + deg[:, 1:2] + 1.0
    odeg = deg[:, 2:3] + deg[:, 3:4] + 1.0
    a = jnp.exp(-_ALPHA * jnp.log(ideg))
    be = jnp.exp(-_BETA * jnp.log(odeg))
    a_ref[...] = a
    be_ref[...] = be
    xbe_ref[...] = x_ref[...] * be


_TCB = 1024


def _tc_prep(deg4, x_pad):
    grid = (_NPAD // _TCB,)
    return pl.pallas_call(
        _tc_prep_body,
        grid=grid,
        in_specs=[
            pl.BlockSpec((_TCB, 4), lambda i: (i, 0)),
            pl.BlockSpec((_TCB, _IN), lambda i: (i, 0)),
        ],
        out_specs=[
            pl.BlockSpec((_TCB, 1), lambda i: (i, 0)),
            pl.BlockSpec((_TCB, 1), lambda i: (i, 0)),
            pl.BlockSpec((_TCB, _IN), lambda i: (i, 0)),
        ],
        out_shape=[
            jax.ShapeDtypeStruct((_NPAD, 1), jnp.float32),
            jax.ShapeDtypeStruct((_NPAD, 1), jnp.float32),
            jax.ShapeDtypeStruct((_NPAD, _IN), jnp.float32),
        ],
    )(deg4, x_pad)


def _tc_mid_body(y1p_ref, xbe_ref, a_ref, be_ref, sbe_ref,
                 ws1t_ref, wt1t_ref, bs1_ref, bt1_ref,
                 cat_ref, d1_ref):
    a = a_ref[...]
    be = be_ref[...]
    y1 = a * (y1p_ref[0] + y1p_ref[1] + xbe_ref[...])
    sbe = sbe_ref[...]
    d1 = a * (sbe[:, 0:1] + sbe[:, 1:2] + be)
    s_h = jnp.maximum(
        jnp.dot(y1, ws1t_ref[...], preferred_element_type=jnp.float32)
        + d1 * bs1_ref[...], 0.0)
    t_h = jnp.maximum(
        jnp.dot(y1, wt1t_ref[...], preferred_element_type=jnp.float32)
        + d1 * bt1_ref[...], 0.0)
    cat_ref[...] = be * jnp.concatenate([t_h, s_h], axis=1)
    d1_ref[...] = d1


def _tc_mid(y1p, xbe, a1, be1, sbeT, ws1t, wt1t, bs1, bt1):
    grid = (_NPAD // _TCB,)
    full = lambda i: (0, 0)
    return pl.pallas_call(
        _tc_mid_body,
        grid=grid,
        in_specs=[
            pl.BlockSpec((_NC, _TCB, _IN), lambda i: (0, i, 0)),
            pl.BlockSpec((_TCB, _IN), lambda i: (i, 0)),
            pl.BlockSpec((_TCB, 1), lambda i: (i, 0)),
            pl.BlockSpec((_TCB, 1), lambda i: (i, 0)),
            pl.BlockSpec((_TCB, 2), lambda i: (i, 0)),
            pl.BlockSpec((_IN, _HID), full),
            pl.BlockSpec((_IN, _HID), full),
            pl.BlockSpec((1, _HID), full),
            pl.BlockSpec((1, _HID), full),
        ],
        out_specs=[
            pl.BlockSpec((_TCB, _IN), lambda i: (i, 0)),
            pl.BlockSpec((_TCB, 1), lambda i: (i, 0)),
        ],
        out_shape=[
            jax.ShapeDtypeStruct((_NPAD, _IN), jnp.float32),
            jax.ShapeDtypeStruct((_NPAD, 1), jnp.float32),
        ],
    )(y1p, xbe, a1, be1, sbeT, ws1t, wt1t, bs1, bt1)


def _tc_final_body(y2p_ref, cat_ref, a_ref, d1_ref,
                   ws2t_ref, wt2t_ref, bs2_ref, bt2_ref, out_ref):
    a = a_ref[...]
    d1 = d1_ref[...]
    y2 = a * (y2p_ref[0] + y2p_ref[1] + cat_ref[...])
    s = (jnp.dot(y2[:, :_HID], ws2t_ref[...],
                 preferred_element_type=jnp.float32) + d1 * bs2_ref[...])
    t = (jnp.dot(y2[:, _HID:], wt2t_ref[...],
                 preferred_element_type=jnp.float32) + d1 * bt2_ref[...])
    out_ref[...] = jnp.concatenate([s, t], axis=1)


def _tc_final(y2p, cat_be, a1, d1, ws2t, wt2t, bs2, bt2):
    grid = (_NPAD // _TCB,)
    full = lambda i: (0, 0)
    return pl.pallas_call(
        _tc_final_body,
        grid=grid,
        in_specs=[
            pl.BlockSpec((_NC, _TCB, _IN), lambda i: (0, i, 0)),
            pl.BlockSpec((_TCB, _IN), lambda i: (i, 0)),
            pl.BlockSpec((_TCB, 1), lambda i: (i, 0)),
            pl.BlockSpec((_TCB, 1), lambda i: (i, 0)),
            pl.BlockSpec((_HID, _OUT), full),
            pl.BlockSpec((_HID, _OUT), full),
            pl.BlockSpec((1, _OUT), full),
            pl.BlockSpec((1, _OUT), full),
        ],
        out_specs=pl.BlockSpec((_TCB, 2 * _OUT), lambda i: (i, 0)),
        out_shape=jax.ShapeDtypeStruct((_NPAD, 2 * _OUT), jnp.float32),
    )(y2p, cat_be, a1, d1, ws2t, wt2t, bs2, bt2)


def kernel(x, edge_index, edge_attr, W_s1, b_s1, W_t1, b_t1,
           W_s2, b_s2, W_t2, b_t2):
    row = edge_index[0]
    col = edge_index[1]
    npad = _EPAD - _E
    dummy = jnp.full((npad,), _NPAD - 1, jnp.int32)
    row3 = jnp.concatenate([row, dummy]).reshape(_NW, _CPW, _CHUNK)
    col3 = jnp.concatenate([col, dummy]).reshape(_NW, _CPW, _CHUNK)

    ideg_p, odeg_p = _sc_degrees(row3, col3)
    deg4 = jnp.stack([ideg_p[0], ideg_p[1], odeg_p[0], odeg_p[1]], axis=1)
    x_pad = jnp.pad(x, ((0, _NPAD - _N), (0, 0)))
    a1, be1, xbe = _tc_prep(deg4, x_pad)

    y1p, sbep = _sc_agg1(xbe, row3, col3, be1.reshape(_NPAD))
    sbeT = jnp.transpose(sbep)

    cat_be, d1 = _tc_mid(y1p, xbe, a1, be1, sbeT,
                         W_s1.T, W_t1.T, b_s1[None, :], b_t1[None, :])

    (y2p,) = _sc_agg2(cat_be, row3, col3)

    outp = _tc_final(y2p, cat_be, a1, d1,
                     W_s2.T, W_t2.T, b_s2[None, :], b_t2[None, :])
    return outp[:_N]


# speedup vs baseline: 21.5360x; 1.2613x over previous
import functools

import jax
import jax.numpy as jnp
from jax import lax
from jax.experimental import pallas as pl
from jax.experimental.pallas import tpu as pltpu
from jax.experimental.pallas import tpu_sc as plsc

_N = 10000
_E = 320000
_IN = 128
_HID = 64
_OUT = 32
_ALPHA = 0.2
_BETA = 0.8

_NC, _NS = 2, 16
_NW = _NC * _NS
_CHUNK = 112
_CPW = 90
_EPAD = _NW * _CPW * _CHUNK
_NPAD = 10240
_STRIPE = _NPAD // _NS

_mesh = plsc.VectorSubcoreMesh(
    core_axis_name="c", subcore_axis_name="s", num_cores=_NC, num_subcores=_NS)


@functools.partial(
    pl.kernel,
    out_type=(jax.ShapeDtypeStruct((_NC, _NPAD), jnp.float32),
              jax.ShapeDtypeStruct((_NC, _NPAD), jnp.float32)),
    mesh=_mesh,
    scratch_types=[
        pltpu.VMEM((_CPW, 2, _CHUNK), jnp.int32),
        pltpu.VMEM((_CHUNK,), jnp.float32),
        pltpu.VMEM((_CHUNK,), jnp.float32),
        pltpu.VMEM_SHARED((_NPAD,), jnp.float32),
        pltpu.VMEM_SHARED((_NPAD,), jnp.float32),
    ],
)
def _sc_degrees(rc_hbm, ideg_out, odeg_out, idx_v, ones_v, zeros_v, iacc, oacc):
    c = lax.axis_index("c")
    s = lax.axis_index("s")
    wid = c * _NS + s

    def initbuf(g, carry):
        ones_v[pl.ds(g * 16, 16)] = jnp.ones((16,), jnp.float32)
        zeros_v[pl.ds(g * 16, 16)] = jnp.zeros((16,), jnp.float32)
        return carry
    lax.fori_loop(0, _CHUNK // 16, initbuf, None)

    def zloop(b, carry):
        off = s * _STRIPE + b * 64
        pltpu.sync_copy(zeros_v.at[pl.ds(0, 64)], iacc.at[pl.ds(off, 64)])
        pltpu.sync_copy(zeros_v.at[pl.ds(0, 64)], oacc.at[pl.ds(off, 64)])
        return carry
    lax.fori_loop(0, _STRIPE // 64, zloop, None)

    pltpu.sync_copy(rc_hbm.at[wid], idx_v)
    plsc.subcore_barrier()

    def body(i, carry):
        pltpu.sync_copy(ones_v, iacc.at[idx_v.at[i, 1]], add=True)
        pltpu.sync_copy(ones_v, oacc.at[idx_v.at[i, 0]], add=True)
        return carry
    lax.fori_loop(0, _CPW, body, None)

    plsc.subcore_barrier()
    sl = pl.ds(s * _STRIPE, _STRIPE)
    pltpu.sync_copy(iacc.at[sl], ideg_out.at[c, sl])
    pltpu.sync_copy(oacc.at[sl], odeg_out.at[c, sl])


def _make_sc_agg(with_sbe):
    D = _IN
    out_type = [jax.ShapeDtypeStruct((_NC, _NPAD, D), jnp.float32)]
    scratch = [
        pltpu.VMEM((2, _CHUNK), jnp.int32),
        pltpu.VMEM((2, _CHUNK), jnp.int32),
        pltpu.VMEM((_CHUNK, D), jnp.float32),
        pltpu.VMEM((_CHUNK, D), jnp.float32),
        pltpu.VMEM_SHARED((_NPAD, D), jnp.float32),
        pltpu.SemaphoreType.DMA,
        pltpu.SemaphoreType.DMA,
        pltpu.SemaphoreType.DMA,
        pltpu.SemaphoreType.DMA,
    ]
    if with_sbe:
        out_type.append(jax.ShapeDtypeStruct((_NC, _NPAD), jnp.float32))
        scratch += [
            pltpu.VMEM((_CHUNK,), jnp.float32),
            pltpu.VMEM((_CHUNK,), jnp.float32),
            pltpu.VMEM_SHARED((_NPAD,), jnp.float32),
        ]

    def body(*refs):
        if with_sbe:
            (feat_hbm, rc_hbm, be_hbm, out_hbm, sbe_out,
             ia, ib, fa, fb, acc, sema, semb, semia, semib,
             bea, beb, sbe_acc) = refs
        else:
            (feat_hbm, rc_hbm, out_hbm,
             ia, ib, fa, fb, acc, sema, semb, semia, semib) = refs
        c = lax.axis_index("c")
        s = lax.axis_index("s")
        wid = c * _NS + s

        def zrow(r, carry):
            for g in range(D // 16):
                fa[r, pl.ds(g * 16, 16)] = jnp.zeros((16,), jnp.float32)
            return carry
        lax.fori_loop(0, _CHUNK, zrow, None)

        def zloop(b, carry):
            pltpu.sync_copy(fa.at[pl.ds(0, 64)],
                            acc.at[pl.ds(s * _STRIPE + b * 64, 64)])
            return carry
        lax.fori_loop(0, _STRIPE // 64, zloop, None)
        if with_sbe:
            def zloop2(b, carry):
                pltpu.sync_copy(fa.at[0],
                                sbe_acc.at[pl.ds(s * _STRIPE + b * D, D)])
                return carry
            lax.fori_loop(0, _STRIPE // D, zloop2, None)
        plsc.subcore_barrier()

        def start(ibuf, fbuf, bbuf, sem):
            pltpu.async_copy(feat_hbm.at[ibuf.at[0]], fbuf, sem)
            if with_sbe:
                pltpu.async_copy(be_hbm.at[ibuf.at[0]], bbuf, sem)

        def gwait(ibuf, fbuf, bbuf, sem):
            pltpu.make_async_copy(feat_hbm.at[ibuf.at[0]], fbuf, sem).wait()
            if with_sbe:
                pltpu.make_async_copy(be_hbm.at[ibuf.at[0]], bbuf, sem).wait()

        def scat(ibuf, fbuf, bbuf):
            pltpu.sync_copy(fbuf, acc.at[ibuf.at[1]], add=True)
            if with_sbe:
                pltpu.sync_copy(bbuf, sbe_acc.at[ibuf.at[1]], add=True)

        def idx_start(k, ibuf, sem):
            pltpu.async_copy(rc_hbm.at[wid, k], ibuf, sem)

        def idx_wait(k, ibuf, sem):
            pltpu.make_async_copy(rc_hbm.at[wid, k], ibuf, sem).wait()

        bea_ = bea if with_sbe else None
        beb_ = beb if with_sbe else None

        pltpu.sync_copy(rc_hbm.at[wid, 0], ia)
        start(ia, fa, bea_, sema)
        pltpu.sync_copy(rc_hbm.at[wid, 1], ib)

        def pair(p, carry):
            k0 = 2 * p
            ka = lax.min(k0 + 2, _CPW - 1)
            kb = lax.min(k0 + 3, _CPW - 1)
            start(ib, fb, beb_, semb)
            gwait(ia, fa, bea_, sema)
            scat(ia, fa, bea_)
            idx_start(ka, ia, semia)
            gwait(ib, fb, beb_, semb)
            idx_wait(ka, ia, semia)
            start(ia, fa, bea_, sema)
            scat(ib, fb, beb_)
            idx_start(kb, ib, semib)
            idx_wait(kb, ib, semib)
            return carry
        lax.fori_loop(0, _CPW // 2, pair, None)

        pltpu.make_async_copy(feat_hbm.at[ia.at[0]], fa, sema).wait()
        if with_sbe:
            pltpu.make_async_copy(be_hbm.at[ia.at[0]], bea, sema).wait()

        plsc.subcore_barrier()
        sl = pl.ds(s * _STRIPE, _STRIPE)
        pltpu.sync_copy(acc.at[sl], out_hbm.at[c, sl])
        if with_sbe:
            pltpu.sync_copy(sbe_acc.at[sl], sbe_out.at[c, sl])

    return pl.kernel(body, out_type=tuple(out_type), mesh=_mesh,
                     scratch_types=scratch)


_sc_agg1 = _make_sc_agg(with_sbe=True)
_sc_agg2 = _make_sc_agg(with_sbe=False)


def _tc_prep_body(deg_ref, x_ref, a_ref, be_ref, xbe_ref):
    deg = deg_ref[...]
    ideg = deg[:, 0:1] + deg[:, 1:2] + 1.0
    odeg = deg[:, 2:3] + deg[:, 3:4] + 1.0
    a = jnp.exp(-_ALPHA * jnp.log(ideg))
    be = jnp.exp(-_BETA * jnp.log(odeg))
    a_ref[...] = a
    be_ref[...] = be
    xbe_ref[...] = x_ref[...] * be


_TCB = 1024


def _tc_prep(deg4, x_pad):
    grid = (_NPAD // _TCB,)
    return pl.pallas_call(
        _tc_prep_body,
        grid=grid,
        in_specs=[
            pl.BlockSpec((_TCB, 4), lambda i: (i, 0)),
            pl.BlockSpec((_TCB, _IN), lambda i: (i, 0)),
        ],
        out_specs=[
            pl.BlockSpec((_TCB, 1), lambda i: (i, 0)),
            pl.BlockSpec((_TCB, 1), lambda i: (i, 0)),
            pl.BlockSpec((_TCB, _IN), lambda i: (i, 0)),
        ],
        out_shape=[
            jax.ShapeDtypeStruct((_NPAD, 1), jnp.float32),
            jax.ShapeDtypeStruct((_NPAD, 1), jnp.float32),
            jax.ShapeDtypeStruct((_NPAD, _IN), jnp.float32),
        ],
    )(deg4, x_pad)


def _tc_mid_body(y1p_ref, xbe_ref, a_ref, be_ref, sbe_ref,
                 ws1t_ref, wt1t_ref, bs1_ref, bt1_ref,
                 cat_ref, d1_ref):
    a = a_ref[...]
    be = be_ref[...]
    y1 = a * (y1p_ref[0] + y1p_ref[1] + xbe_ref[...])
    sbe = sbe_ref[...]
    d1 = a * (sbe[:, 0:1] + sbe[:, 1:2] + be)
    s_h = jnp.maximum(
        jnp.dot(y1, ws1t_ref[...], preferred_element_type=jnp.float32)
        + d1 * bs1_ref[...], 0.0)
    t_h = jnp.maximum(
        jnp.dot(y1, wt1t_ref[...], preferred_element_type=jnp.float32)
        + d1 * bt1_ref[...], 0.0)
    cat_ref[...] = be * jnp.concatenate([t_h, s_h], axis=1)
    d1_ref[...] = d1


def _tc_mid(y1p, xbe, a1, be1, sbeT, ws1t, wt1t, bs1, bt1):
    grid = (_NPAD // _TCB,)
    full = lambda i: (0, 0)
    return pl.pallas_call(
        _tc_mid_body,
        grid=grid,
        in_specs=[
            pl.BlockSpec((_NC, _TCB, _IN), lambda i: (0, i, 0)),
            pl.BlockSpec((_TCB, _IN), lambda i: (i, 0)),
            pl.BlockSpec((_TCB, 1), lambda i: (i, 0)),
            pl.BlockSpec((_TCB, 1), lambda i: (i, 0)),
            pl.BlockSpec((_TCB, 2), lambda i: (i, 0)),
            pl.BlockSpec((_IN, _HID), full),
            pl.BlockSpec((_IN, _HID), full),
            pl.BlockSpec((1, _HID), full),
            pl.BlockSpec((1, _HID), full),
        ],
        out_specs=[
            pl.BlockSpec((_TCB, _IN), lambda i: (i, 0)),
            pl.BlockSpec((_TCB, 1), lambda i: (i, 0)),
        ],
        out_shape=[
            jax.ShapeDtypeStruct((_NPAD, _IN), jnp.float32),
            jax.ShapeDtypeStruct((_NPAD, 1), jnp.float32),
        ],
    )(y1p, xbe, a1, be1, sbeT, ws1t, wt1t, bs1, bt1)


def _tc_final_body(y2p_ref, cat_ref, a_ref, d1_ref,
                   ws2t_ref, wt2t_ref, bs2_ref, bt2_ref, out_ref):
    a = a_ref[...]
    d1 = d1_ref[...]
    y2 = a * (y2p_ref[0] + y2p_ref[1] + cat_ref[...])
    s = (jnp.dot(y2[:, :_HID], ws2t_ref[...],
                 preferred_element_type=jnp.float32) + d1 * bs2_ref[...])
    t = (jnp.dot(y2[:, _HID:], wt2t_ref[...],
                 preferred_element_type=jnp.float32) + d1 * bt2_ref[...])
    out_ref[...] = jnp.concatenate([s, t], axis=1)


def _tc_final(y2p, cat_be, a1, d1, ws2t, wt2t, bs2, bt2):
    grid = (_NPAD // _TCB,)
    full = lambda i: (0, 0)
    return pl.pallas_call(
        _tc_final_body,
        grid=grid,
        in_specs=[
            pl.BlockSpec((_NC, _TCB, _IN), lambda i: (0, i, 0)),
            pl.BlockSpec((_TCB, _IN), lambda i: (i, 0)),
            pl.BlockSpec((_TCB, 1), lambda i: (i, 0)),
            pl.BlockSpec((_TCB, 1), lambda i: (i, 0)),
            pl.BlockSpec((_HID, _OUT), full),
            pl.BlockSpec((_HID, _OUT), full),
            pl.BlockSpec((1, _OUT), full),
            pl.BlockSpec((1, _OUT), full),
        ],
        out_specs=pl.BlockSpec((_TCB, 2 * _OUT), lambda i: (i, 0)),
        out_shape=jax.ShapeDtypeStruct((_NPAD, 2 * _OUT), jnp.float32),
    )(y2p, cat_be, a1, d1, ws2t, wt2t, bs2, bt2)


def kernel(x, edge_index, edge_attr, W_s1, b_s1, W_t1, b_t1,
           W_s2, b_s2, W_t2, b_t2):
    row = edge_index[0]
    col = edge_index[1]
    npad = _EPAD - _E
    dummy = jnp.full((npad,), _NPAD - 1, jnp.int32)
    row_p = jnp.concatenate([row, dummy]).reshape(_NW, _CPW, _CHUNK)
    col_p = jnp.concatenate([col, dummy]).reshape(_NW, _CPW, _CHUNK)
    rc4 = jnp.stack([row_p, col_p], axis=2)

    ideg_p, odeg_p = _sc_degrees(rc4)
    deg4 = jnp.stack([ideg_p[0], ideg_p[1], odeg_p[0], odeg_p[1]], axis=1)
    x_pad = jnp.pad(x, ((0, _NPAD - _N), (0, 0)))
    a1, be1, xbe = _tc_prep(deg4, x_pad)

    y1p, sbep = _sc_agg1(xbe, rc4, be1.reshape(_NPAD))
    sbeT = jnp.transpose(sbep)

    cat_be, d1 = _tc_mid(y1p, xbe, a1, be1, sbeT,
                         W_s1.T, W_t1.T, b_s1[None, :], b_t1[None, :])

    (y2p,) = _sc_agg2(cat_be, rc4)

    outp = _tc_final(y2p, cat_be, a1, d1,
                     W_s2.T, W_t2.T, b_s2[None, :], b_t2[None, :])
    return outp[:_N]
